# baseline (speedup 1.0000x reference)
import functools

import jax
import jax.numpy as jnp
from jax import lax
from jax.experimental import pallas as pl
from jax.experimental.pallas import tpu as pltpu
from jax.experimental.pallas import tpu_sc as plsc

BVS = 1000000
BD = 64
MOD = BVS - 1
ROW = 4096
NROWS = 4
B = NROWS * ROW
L = 16

_info = plsc.get_sparse_core_info()
NC = _info.num_cores
NS = _info.num_subcores
NW = NC * NS
BPW = B // NW
NV = BPW // L
NG = 4
GR = BPW // NG

_mesh = plsc.VectorSubcoreMesh(core_axis_name="c", subcore_axis_name="s")


@functools.partial(
    pl.kernel,
    mesh=_mesh,
    out_type=jax.ShapeDtypeStruct((B, BD), jnp.float32),
    scratch_types=[
        pltpu.VMEM((8 + BPW,), jnp.int32),
        pltpu.VMEM((BPW,), jnp.int32),
        pltpu.VMEM((NG, GR), jnp.int32),
        pltpu.VMEM((2, GR, 2 * BD), jnp.float32),
        pltpu.VMEM((BPW, BD), jnp.float32),
        pltpu.VMEM((L,), jnp.float32),
        pltpu.SemaphoreType.DMA,
    ],
)
def _bigram_gather(ids_hbm, emb2_hbm, scale_hbm, out_hbm,
                   ids_v, oidx_v, gidx_v, pairs_v, outst_v, scale_v, sem):
    wid = lax.axis_index("s") * NC + lax.axis_index("c")
    base = wid * BPW

    pltpu.sync_copy(scale_hbm, scale_v)
    pltpu.sync_copy(ids_hbm.at[pl.ds(base, BPW)], ids_v.at[pl.ds(8, BPW)])

    @pl.when(wid != 0)
    def _():
        pltpu.sync_copy(ids_hbm.at[pl.ds(base - 8, 8)], ids_v.at[pl.ds(0, 8)])

    iota = lax.iota(jnp.int32, L)

    def _hash16(i, carry):
        cur = ids_v[pl.ds(8 + i * L, L)]
        prev = ids_v[pl.ds(7 + i * L, L)]
        h = lax.rem(lax.bitwise_xor(cur * 36313, prev * 27191), MOD)
        gpos = base + i * L + iota
        o = jnp.where((gpos & (ROW - 1)) == 0, MOD, h)
        oidx_v[pl.ds(i * L, L)] = o
        gidx_v[i // (GR // L), pl.ds((i % (GR // L)) * L, L)] = (
            lax.shift_right_logical(o, 1))
        return carry

    lax.fori_loop(0, NV, _hash16, 0)

    scale_vec = scale_v[...]

    def _start_gather(j):
        return pltpu.async_copy(emb2_hbm.at[gidx_v.at[j]],
                                pairs_v.at[j % 2], sem)

    def _make_select(j):
        def _select16(i, carry):
            gbase = j * GR + i * L
            half = (oidx_v[pl.ds(gbase, L)] & 1) * BD
            for jj in range(L):
                p = half[jj]
                lr = i * L + jj
                for c in range(BD // L):
                    outst_v[gbase + jj, pl.ds(c * L, L)] = (
                        pairs_v[j % 2, lr, pl.ds(p + c * L, L)] * scale_vec)
            return carry
        return _select16

    handle = _start_gather(0)
    for j in range(NG):
        nxt = _start_gather(j + 1) if j + 1 < NG else None
        handle.wait()
        lax.fori_loop(0, GR // L, _make_select(j), 0)
        handle = nxt

    pltpu.sync_copy(outst_v, out_hbm.at[pl.ds(base, BPW)])


def kernel(ids, emb, scale):
    ids_flat = ids.reshape(-1)
    emb2 = emb.reshape(BVS // 2, 2 * BD)
    scale16 = jnp.broadcast_to(scale.astype(jnp.float32).reshape(()), (L,))
    out = _bigram_gather(ids_flat, emb2, scale16)
    return out.reshape(ids.shape + (BD,))


# speedup vs baseline: 1.7209x; 1.7209x over previous
import functools

import jax
import jax.numpy as jnp
from jax import lax
from jax.experimental import pallas as pl
from jax.experimental.pallas import tpu as pltpu
from jax.experimental.pallas import tpu_sc as plsc

BVS = 1000000
BD = 64
MOD = BVS - 1
ROW = 4096
NROWS = 4
B = NROWS * ROW
L = 16

_info = plsc.get_sparse_core_info()
NC = _info.num_cores
NS = _info.num_subcores
NW = NC * NS
BPW = B // NW
NV = BPW // L

_mesh = plsc.VectorSubcoreMesh(core_axis_name="c", subcore_axis_name="s")


@functools.partial(
    pl.kernel,
    mesh=_mesh,
    out_type=jax.ShapeDtypeStruct((B, BD), jnp.float32),
    scratch_types=[
        pltpu.VMEM((8 + BPW,), jnp.int32),
        pltpu.VMEM((BPW, BD), jnp.float32),
        pltpu.VMEM((L,), jnp.float32),
        pltpu.SemaphoreType.DMA,
    ],
)
def _bigram_gather(ids_hbm, emb_hbm, scale_hbm, out_hbm,
                   ids_v, rows_v, scale_v, sem):
    wid = lax.axis_index("s") * NC + lax.axis_index("c")
    base = wid * BPW

    pltpu.sync_copy(scale_hbm, scale_v)
    pltpu.sync_copy(ids_hbm.at[pl.ds(base, BPW)], ids_v.at[pl.ds(8, BPW)])

    @pl.when(wid != 0)
    def _():
        pltpu.sync_copy(ids_hbm.at[pl.ds(base - 8, 8)], ids_v.at[pl.ds(0, 8)])

    iota = lax.iota(jnp.int32, L)

    def _gather16(i, carry):
        cur = ids_v[pl.ds(8 + i * L, L)]
        prev = ids_v[pl.ds(7 + i * L, L)]
        h = lax.rem(lax.bitwise_xor(cur * 36313, prev * 27191), MOD)
        gpos = base + i * L + iota
        o = jnp.where((gpos & (ROW - 1)) == 0, MOD, h)
        rbase = i * L
        for j in range(L):
            pltpu.async_copy(emb_hbm.at[o[j]], rows_v.at[rbase + j], sem)
        return carry

    lax.fori_loop(0, NV, _gather16, 0)

    pltpu.make_async_copy(out_hbm.at[pl.ds(base, BPW)], rows_v, sem).wait()

    scale_vec = scale_v[...]

    def _scale_rows(r, carry):
        for c in range(BD // L):
            rows_v[r, pl.ds(c * L, L)] = rows_v[r, pl.ds(c * L, L)] * scale_vec
        return carry

    lax.fori_loop(0, BPW, _scale_rows, 0)

    pltpu.sync_copy(rows_v, out_hbm.at[pl.ds(base, BPW)])


def kernel(ids, emb, scale):
    ids_flat = ids.reshape(-1)
    scale16 = jnp.broadcast_to(scale.astype(jnp.float32).reshape(()), (L,))
    out = _bigram_gather(ids_flat, emb, scale16)
    return out.reshape(ids.shape + (BD,))
